# baseline (speedup 1.0000x reference)
import functools

import jax
import jax.numpy as jnp
from jax import lax
from jax.experimental import pallas as pl
from jax.experimental.pallas import tpu as pltpu
from jax.experimental.pallas import tpu_sc as plsc

C = 128
N = 10000
K = 32
NUM_CORES = 2
NUM_SUBCORES = 16
NUM_WORKERS = NUM_CORES * NUM_SUBCORES
N_PAD = 10240
PER_WORKER = N_PAD // NUM_WORKERS
NB = 64
NCHUNKS = PER_WORKER // NB
LANES = 16

STAGE_ROWS = 632
STAGE_LAST = N - 15 * STAGE_ROWS


NB_BIG = 128
NB_SMALL = PER_WORKER - 2 * NB_BIG


def _sc_gather_sum(xt, idx_big, idx_small):
  mesh = plsc.VectorSubcoreMesh(
      core_axis_name="c", subcore_axis_name="s")

  @functools.partial(
      pl.kernel,
      mesh=mesh,
      out_type=jax.ShapeDtypeStruct((N_PAD, C), jnp.float32),
      scratch_types=[
          pltpu.VMEM((2 * K, NB_BIG), jnp.int32),
          pltpu.VMEM((K, NB_SMALL), jnp.int32),
          pltpu.VMEM((NB_BIG, C), jnp.float32),
          pltpu.VMEM((NB_BIG, C), jnp.float32),
          pltpu.VMEM_SHARED((N, C), jnp.float32),
          pltpu.SemaphoreType.DMA,
          pltpu.SemaphoreType.DMA,
          pltpu.SemaphoreType.DMA,
      ],
  )
  def body(xt_hbm, idxb_hbm, idxs_hbm, out_hbm, idxb, idxs,
           acc0, acc1, tbl_s, sem_a, sem_b, sem_c):
    sid = lax.axis_index("s")
    wid = sid * NUM_CORES + lax.axis_index("c")
    base = wid * PER_WORKER

    @pl.when(sid < 15)
    def _stage_main():
      off = pl.multiple_of(sid * STAGE_ROWS, 8)
      pltpu.sync_copy(xt_hbm.at[pl.ds(off, STAGE_ROWS)],
                      tbl_s.at[pl.ds(off, STAGE_ROWS)])

    @pl.when(sid == 15)
    def _stage_last():
      pltpu.sync_copy(xt_hbm.at[pl.ds(15 * STAGE_ROWS, STAGE_LAST)],
                      tbl_s.at[pl.ds(15 * STAGE_ROWS, STAGE_LAST)])

    pltpu.sync_copy(idxb_hbm.at[wid], idxb)
    pltpu.sync_copy(idxs_hbm.at[wid], idxs)

    plsc.subcore_barrier()

    zv = jnp.zeros((LANES,), jnp.float32)

    def zero(acc, rows):
      @pl.loop(0, rows)
      def _z(r):
        for cs in range(C // LANES):
          acc[r, pl.ds(cs * LANES, LANES)] = zv

    def fire(idx_t, row0, acc, sem):
      @pl.loop(0, K)
      def _f(k):
        pltpu.async_copy(tbl_s.at[idx_t.at[row0 + k]], acc, sem, add=True)

    def drain(idx_t, acc, sem):
      @pl.loop(0, K)
      def _d(k):
        pltpu.make_async_copy(tbl_s.at[idx_t.at[0]], acc, sem).wait()

    acc2 = acc0.at[pl.ds(0, NB_SMALL)]
    zero(acc0, NB_BIG)
    fire(idxb, 0, acc0, sem_a)
    zero(acc1, NB_BIG)
    fire(idxb, K, acc1, sem_b)
    drain(idxb, acc0, sem_a)
    pltpu.sync_copy(acc0, out_hbm.at[pl.ds(base, NB_BIG)])
    zero(acc2, NB_SMALL)
    fire(idxs, 0, acc2, sem_c)
    drain(idxb, acc1, sem_b)
    pltpu.sync_copy(acc1, out_hbm.at[pl.ds(base + NB_BIG, NB_BIG)])
    drain(idxs, acc2, sem_c)
    pltpu.sync_copy(acc2, out_hbm.at[pl.ds(base + 2 * NB_BIG, NB_SMALL)])

  return body(xt, idx_big, idx_small)


BN = 512


def _tc_self(x2d, w, b1d, eps2d):
  def body(eps_ref, w_ref, b_ref, x_ref, o_ref):
    scale = 1.0 + eps_ref[0, 0]
    o_ref[...] = lax.dot_general(
        x_ref[...] * scale, w_ref[...],
        dimension_numbers=(((0,), (1,)), ((), ())),
        preferred_element_type=jnp.float32,
    ) + b_ref[...]

  grid = (pl.cdiv(N, BN),)
  return pl.pallas_call(
      body,
      grid=grid,
      in_specs=[
          pl.BlockSpec((1, 1), lambda i: (0, 0)),
          pl.BlockSpec((C, C), lambda i: (0, 0)),
          pl.BlockSpec((1, C), lambda i: (0, 0)),
          pl.BlockSpec((C, BN), lambda i: (0, i)),
      ],
      out_specs=pl.BlockSpec((BN, C), lambda i: (i, 0)),
      out_shape=jax.ShapeDtypeStruct((N, C), jnp.float32),
  )(eps2d, w, b1d, x2d)


def _tc_neigh(t1t, s, w):
  def body(w_ref, t1_ref, s_ref, o_ref):
    t2 = lax.dot_general(
        s_ref[...], w_ref[...],
        dimension_numbers=(((1,), (1,)), ((), ())),
        preferred_element_type=jnp.float32,
    )
    o_ref[...] = jnp.maximum(t1_ref[...] + t2, 0.0)

  grid = (pl.cdiv(N, BN),)
  return pl.pallas_call(
      body,
      grid=grid,
      in_specs=[
          pl.BlockSpec((C, C), lambda i: (0, 0)),
          pl.BlockSpec((BN, C), lambda i: (i, 0)),
          pl.BlockSpec((BN, C), lambda i: (i, 0)),
      ],
      out_specs=pl.BlockSpec((BN, C), lambda i: (i, 0)),
      out_shape=jax.ShapeDtypeStruct((N, C), jnp.float32),
  )(w, t1t, s)


def kernel(x, edge_index, W, b, eps):
  x2d = x.reshape(C, N)
  xt = x2d.T
  idx = edge_index[0].reshape(N, K)
  pad_idx = (jnp.arange((N_PAD - N) * K, dtype=jnp.int32) % N).reshape(
      N_PAD - N, K)
  idx_all = jnp.concatenate([idx, pad_idx], axis=0).reshape(
      NUM_WORKERS, PER_WORKER, K)
  idx_big = (
      idx_all[:, :2 * NB_BIG]
      .reshape(NUM_WORKERS, 2, NB_BIG, K)
      .transpose(0, 1, 3, 2)
      .reshape(NUM_WORKERS, 2 * K, NB_BIG)
  )
  idx_small = idx_all[:, 2 * NB_BIG:].transpose(0, 2, 1)

  s = _sc_gather_sum(xt, idx_big, idx_small)

  b1d = b.reshape(1, C)
  eps2d = eps.reshape(1, 1)
  t1t = _tc_self(x2d, W, b1d, eps2d)
  out_t = _tc_neigh(t1t, s, W)
  return out_t.T.reshape(1, C, N, 1)
